# baseline (speedup 1.0000x reference)
import functools

import jax
import jax.numpy as jnp
from jax import lax
from jax.experimental import pallas as pl
from jax.experimental.pallas import tpu as pltpu
from jax.experimental.pallas import tpu_sc as plsc

N = 16384
D = 128
K = 8192

TB = 1024
KB = 1024

NC = 2
NS = 16
NW = NC * NS
B_PER_W = N // NW
GCH = 4
GW = B_PER_W // GCH


def _dist_argmin_body(flat_ref, en_ref, w_ref, wn_ref, ids_ref, mval, midx):
    kb = pl.program_id(1)
    f = flat_ref[...]
    w = w_ref[...]
    dot = lax.dot_general(f, w, (((1,), (1,)), ((), ())),
                          preferred_element_type=jnp.float32)
    s = (en_ref[...] + wn_ref[...]) - 2.0 * dot
    bm = jnp.min(s, axis=1, keepdims=True)
    lane = lax.broadcasted_iota(jnp.int32, s.shape, 1)
    bi = jnp.min(jnp.where(s == bm, lane, jnp.int32(K)), axis=1,
                 keepdims=True) + kb * KB

    @pl.when(kb == 0)
    def _():
        mval[...] = bm
        midx[...] = bi

    @pl.when(kb != 0)
    def _():
        upd = bm < mval[...]
        mval[...] = jnp.where(upd, bm, mval[...])
        midx[...] = jnp.where(upd, bi, midx[...])

    @pl.when(kb == pl.num_programs(1) - 1)
    def _():
        ids_ref[...] = midx[...]


def _argmin_ids(flat, en, weight, wn):
    return pl.pallas_call(
        _dist_argmin_body,
        grid=(N // TB, K // KB),
        in_specs=[
            pl.BlockSpec((TB, D), lambda i, k: (i, 0)),
            pl.BlockSpec((TB, 1), lambda i, k: (i, 0)),
            pl.BlockSpec((KB, D), lambda i, k: (k, 0)),
            pl.BlockSpec((1, KB), lambda i, k: (0, k)),
        ],
        out_specs=pl.BlockSpec((TB, 1), lambda i, k: (i, 0)),
        out_shape=jax.ShapeDtypeStruct((N, 1), jnp.int32),
        scratch_shapes=[
            pltpu.VMEM((TB, 1), jnp.float32),
            pltpu.VMEM((TB, 1), jnp.int32),
        ],
    )(flat, en, weight, wn)


@functools.cache
def _make_sc_gather():
    @functools.partial(
        pl.kernel,
        mesh=plsc.VectorSubcoreMesh(core_axis_name="c", subcore_axis_name="s",
                                    num_cores=NC, num_subcores=NS),
        out_type=jax.ShapeDtypeStruct((NW, GCH, GW, D), jnp.float32),
        scratch_types=[
            pltpu.VMEM((GCH, GW), jnp.int32),
            pltpu.VMEM((GCH, GW, D), jnp.float32),
            pltpu.SemaphoreType.DMA,
        ],
    )
    def _sc_gather(table_hbm, idx_hbm, out_hbm, idx_v, rows_v, sem):
        wid = lax.axis_index("s") * NC + lax.axis_index("c")
        pltpu.sync_copy(idx_hbm.at[wid], idx_v)
        copies = [
            pltpu.async_copy(table_hbm.at[idx_v.at[j]], rows_v.at[j], sem)
            for j in range(GCH)
        ]
        for c in copies:
            c.wait()
        pltpu.sync_copy(rows_v, out_hbm.at[wid])

    return _sc_gather


def kernel(encodings, weight):
    b, c, h, w = encodings.shape
    flat = jnp.transpose(encodings, (0, 2, 3, 1)).reshape(-1, c)
    en = jnp.sum(jnp.square(flat), axis=1)[:, None]
    wn = jnp.sum(jnp.square(weight), axis=1)[None, :]
    ids = _argmin_ids(flat, en, weight, wn)
    gathered = _make_sc_gather()(weight, ids.reshape(NW, GCH, GW))
    ids_out = ids.reshape(b, h, w)
    q_out = jnp.transpose(gathered.reshape(b, h, w, c), (0, 3, 1, 2))
    return ids_out, q_out


# speedup vs baseline: 1.7664x; 1.7664x over previous
import functools

import jax
import jax.numpy as jnp
from jax import lax
from jax.experimental import pallas as pl
from jax.experimental.pallas import tpu as pltpu
from jax.experimental.pallas import tpu_sc as plsc

N = 16384
D = 128
K = 8192

T = 1024
CH = 256
HH = CH // 2
NCH = K // CH
BPS = 2

NC = 2
NS = 16
NW = NC * NS
B_PER_W = N // NW
GCH = 4
GW = B_PER_W // GCH


def _dist_argmin_body(x_ref, en_ref, w2_ref, wn_ref, ids_ref):
    p_iota = lax.broadcasted_iota(jnp.int32, (HH, T), 0)
    for sb in range(BPS):
        x = x_ref[sb]
        en = en_ref[sb]
        run_val = None
        run_code = None
        for c in range(NCH):
            w2c = w2_ref[c * CH:(c + 1) * CH, :]
            wnc = wn_ref[c * CH:(c + 1) * CH, :]
            dot2 = lax.dot_general(w2c, x, (((1,), (0,)), ((), ())),
                                   preferred_element_type=jnp.float32)
            s = (en + wnc) + dot2
            lo = s[:HH, :]
            hi = s[HH:, :]
            m1 = hi < lo
            fv = jnp.minimum(lo, hi)
            pc = jnp.where(m1, jnp.int32(2 * c + 1), jnp.int32(2 * c))
            if c == 0:
                run_val = fv
                run_code = pc
            else:
                upd = fv < run_val
                run_val = jnp.minimum(run_val, fv)
                run_code = jnp.where(upd, pc, run_code)
        k_full = run_code * HH + p_iota
        m = jnp.min(run_val, axis=0, keepdims=True)
        ids = jnp.min(jnp.where(run_val == m, k_full, jnp.int32(1 << 30)),
                      axis=0, keepdims=True)
        ids_ref[sb, :, :] = ids


def _argmin_ids(enc3, en3, w2, wn2):
    nb = enc3.shape[0]
    return pl.pallas_call(
        _dist_argmin_body,
        grid=(nb // BPS,),
        in_specs=[
            pl.BlockSpec((BPS, D, T), lambda b: (b, 0, 0)),
            pl.BlockSpec((BPS, 1, T), lambda b: (b, 0, 0)),
            pl.BlockSpec((K, D), lambda b: (0, 0)),
            pl.BlockSpec((K, 1), lambda b: (0, 0)),
        ],
        out_specs=pl.BlockSpec((BPS, 1, T), lambda b: (b, 0, 0)),
        out_shape=jax.ShapeDtypeStruct((nb, 1, T), jnp.int32),
    )(enc3, en3, w2, wn2)


@functools.cache
def _make_sc_gather():
    @functools.partial(
        pl.kernel,
        mesh=plsc.VectorSubcoreMesh(core_axis_name="c", subcore_axis_name="s",
                                    num_cores=NC, num_subcores=NS),
        out_type=jax.ShapeDtypeStruct((NW, GCH, GW, D), jnp.float32),
        scratch_types=[
            pltpu.VMEM((GCH, GW), jnp.int32),
            pltpu.VMEM((GCH, GW, D), jnp.float32),
            pltpu.SemaphoreType.DMA,
        ],
    )
    def _sc_gather(table_hbm, idx_hbm, out_hbm, idx_v, rows_v, sem):
        wid = lax.axis_index("s") * NC + lax.axis_index("c")
        pltpu.sync_copy(idx_hbm.at[wid], idx_v)
        copies = [
            pltpu.async_copy(table_hbm.at[idx_v.at[j]], rows_v.at[j], sem)
            for j in range(GCH)
        ]
        for c in copies:
            c.wait()
        pltpu.sync_copy(rows_v, out_hbm.at[wid])

    return _sc_gather


def kernel(encodings, weight):
    b, c, h, w = encodings.shape
    enc3 = encodings.reshape(b, c, h * w)
    en3 = jnp.sum(jnp.square(enc3), axis=1, keepdims=True)
    wn2 = jnp.sum(jnp.square(weight), axis=1)[:, None]
    w2 = -2.0 * weight
    ids = _argmin_ids(enc3, en3, w2, wn2)
    gathered = _make_sc_gather()(weight, ids.reshape(NW, GCH, GW))
    ids_out = ids.reshape(b, h, w)
    q_out = jnp.transpose(gathered.reshape(b, h, w, c), (0, 3, 1, 2))
    return ids_out, q_out


# speedup vs baseline: 1.7804x; 1.0079x over previous
import functools

import jax
import jax.numpy as jnp
from jax import lax
from jax.experimental import pallas as pl
from jax.experimental.pallas import tpu as pltpu
from jax.experimental.pallas import tpu_sc as plsc

N = 16384
D = 128
K = 8192

T = 1024
CH = 256
HH = CH // 2
NCH = K // CH
BPS = 4

NC = 2
NS = 16
NW = NC * NS
B_PER_W = N // NW
GCH = 4
GW = B_PER_W // GCH


def _dist_argmin_body(x_ref, en_ref, w2_ref, wn_ref, ids_ref):
    p_iota = lax.broadcasted_iota(jnp.int32, (HH, T), 0)
    for sb in range(BPS):
        x = x_ref[sb]
        en = en_ref[sb]
        run_val = None
        run_code = None
        for c in range(NCH):
            w2c = w2_ref[c * CH:(c + 1) * CH, :]
            wnc = wn_ref[c * CH:(c + 1) * CH, :]
            dot2 = lax.dot_general(w2c, x, (((1,), (0,)), ((), ())),
                                   preferred_element_type=jnp.float32)
            s = (en + wnc) + dot2
            lo = s[:HH, :]
            hi = s[HH:, :]
            m1 = hi < lo
            fv = jnp.minimum(lo, hi)
            pc = jnp.where(m1, jnp.int32(2 * c + 1), jnp.int32(2 * c))
            if c == 0:
                run_val = fv
                run_code = pc
            else:
                upd = fv < run_val
                run_val = jnp.minimum(run_val, fv)
                run_code = jnp.where(upd, pc, run_code)
        k_full = run_code * HH + p_iota
        m = jnp.min(run_val, axis=0, keepdims=True)
        ids = jnp.min(jnp.where(run_val == m, k_full, jnp.int32(1 << 30)),
                      axis=0, keepdims=True)
        ids_ref[sb, :, :] = ids


def _argmin_ids(enc3, en3, w2, wn2):
    nb = enc3.shape[0]
    return pl.pallas_call(
        _dist_argmin_body,
        grid=(nb // BPS,),
        in_specs=[
            pl.BlockSpec((BPS, D, T), lambda b: (b, 0, 0)),
            pl.BlockSpec((BPS, 1, T), lambda b: (b, 0, 0)),
            pl.BlockSpec((K, D), lambda b: (0, 0)),
            pl.BlockSpec((K, 1), lambda b: (0, 0)),
        ],
        out_specs=pl.BlockSpec((BPS, 1, T), lambda b: (b, 0, 0)),
        out_shape=jax.ShapeDtypeStruct((nb, 1, T), jnp.int32),
    )(enc3, en3, w2, wn2)


@functools.cache
def _make_sc_gather():
    @functools.partial(
        pl.kernel,
        mesh=plsc.VectorSubcoreMesh(core_axis_name="c", subcore_axis_name="s",
                                    num_cores=NC, num_subcores=NS),
        out_type=jax.ShapeDtypeStruct((NW, GCH, GW, D), jnp.float32),
        scratch_types=[
            pltpu.VMEM((GCH, GW), jnp.int32),
            pltpu.VMEM((GCH, GW, D), jnp.float32),
            pltpu.SemaphoreType.DMA,
        ],
    )
    def _sc_gather(table_hbm, idx_hbm, out_hbm, idx_v, rows_v, sem):
        wid = lax.axis_index("s") * NC + lax.axis_index("c")
        pltpu.sync_copy(idx_hbm.at[wid], idx_v)
        copies = [
            pltpu.async_copy(table_hbm.at[idx_v.at[j]], rows_v.at[j], sem)
            for j in range(GCH)
        ]
        for c in copies:
            c.wait()
        pltpu.sync_copy(rows_v, out_hbm.at[wid])

    return _sc_gather


def kernel(encodings, weight):
    b, c, h, w = encodings.shape
    enc3 = encodings.reshape(b, c, h * w)
    en3 = jnp.sum(jnp.square(enc3), axis=1, keepdims=True)
    wn2 = jnp.sum(jnp.square(weight), axis=1)[:, None]
    w2 = -2.0 * weight
    ids = _argmin_ids(enc3, en3, w2, wn2)
    gathered = _make_sc_gather()(weight, ids.reshape(NW, GCH, GW))
    ids_out = ids.reshape(b, h, w)
    q_out = jnp.transpose(gathered.reshape(b, h, w, c), (0, 3, 1, 2))
    return ids_out, q_out
